# baseline (speedup 1.0000x reference)
import jax
import jax.numpy as jnp
from jax import lax
from jax.experimental import pallas as pl
from jax.experimental.pallas import tpu as pltpu
from jax.experimental.pallas import tpu_sc as plsc

N = 50000
D = 128
E = 200000

NPAD = 50176
EPAD = 200192
CH = 6272
ACCR = 6400
STRIPE = EPAD // 16
HSTRIPE = STRIPE // 2
ESH = EPAD // 32
K = 32
PAIR = 2 * K

_mesh = plsc.VectorSubcoreMesh(core_axis_name="c", subcore_axis_name="s")
_sc_params = pltpu.CompilerParams(needs_layout_passes=False)


def _sig(z):
    return 1.0 / (1.0 + jnp.exp(-z))


def _unpack_src(p16):
    return p16 & 0xFFFF


def _unpack_dst(p16):
    return lax.shift_right_logical(p16, 16)


def _sc_prep_body(src_all, dst_all, cnt, pk, acc_l, se_v, de_v, pk_v, tmp,
                  res, stage):
    c = lax.axis_index("c")
    s = lax.axis_index("s")
    zero16 = jnp.zeros((16,), jnp.float32)
    ones16 = jnp.ones((16,), jnp.float32)

    for a in range(2):
        arr = c * 2 + a

        def _z(i, _):
            for q in range(8):
                acc_l[pl.ds((i * 8 + q) * 16, 16)] = zero16
            return 0
        lax.fori_loop(0, NPAD // 128, _z, 0)

        def _scan(i, _):
            s16 = se_v[pl.ds(i * 16, 16)]
            d16 = de_v[pl.ds(i * 16, 16)]
            pk_v[pl.ds(i * 16, 16)] = (d16 << 16) | s16
            plsc.addupdate_scatter(acc_l, [d16], ones16)
            return 0

        for h in range(2):
            base = arr * EPAD + s * STRIPE + h * HSTRIPE
            pltpu.sync_copy(src_all.at[pl.ds(base, HSTRIPE)], se_v)
            pltpu.sync_copy(dst_all.at[pl.ds(base, HSTRIPE)], de_v)
            lax.fori_loop(0, HSTRIPE // 16, _scan, 0)
            pltpu.sync_copy(pk_v, pk.at[pl.ds(base, HSTRIPE)])

        pltpu.sync_copy(acc_l, stage.at[pl.ds(s * NPAD, NPAD)])
        plsc.subcore_barrier()

        colbase = s * (NPAD // 16)
        pltpu.sync_copy(stage.at[pl.ds(colbase, NPAD // 16)], res)
        for j in range(1, 16):
            pltpu.sync_copy(stage.at[pl.ds(j * NPAD + colbase, NPAD // 16)],
                            tmp)

            def _add(i, _):
                for q in range(4):
                    o = (i * 4 + q) * 16
                    res[pl.ds(o, 16)] = (res[pl.ds(o, 16)]
                                         + tmp[pl.ds(o, 16)])
                return 0
            lax.fori_loop(0, NPAD // 16 // 64, _add, 0)
        pltpu.sync_copy(res, cnt.at[pl.ds(arr * NPAD + colbase, NPAD // 16)])
        plsc.subcore_barrier()


_sc_prep = pl.kernel(
    _sc_prep_body,
    out_type=(jax.ShapeDtypeStruct((4 * NPAD,), jnp.float32),
              jax.ShapeDtypeStruct((4 * EPAD,), jnp.int32)),
    mesh=_mesh,
    compiler_params=_sc_params,
    scratch_types=[
        pltpu.VMEM((NPAD,), jnp.float32),
        pltpu.VMEM((HSTRIPE,), jnp.int32),
        pltpu.VMEM((HSTRIPE,), jnp.int32),
        pltpu.VMEM((HSTRIPE,), jnp.int32),
        pltpu.VMEM((NPAD // 16,), jnp.float32),
        pltpu.VMEM((NPAD // 16,), jnp.float32),
        pltpu.VMEM_SHARED((16 * NPAD,), jnp.float32),
    ],
)


def _sc_scales_body(pk_em, pk_eo, gs, gd, t1, t2, scm, sco,
                    g1_v, g2_v, pk_v, out_v):
    c = lax.axis_index("c")
    s = lax.axis_index("s")
    wid = s * 2 + c
    ebase = wid * ESH

    for part, (ga, gb, pkr, outr) in enumerate(
            (((gs, gd, pk_em, scm)), (t1, t2, pk_eo, sco))):
        pltpu.sync_copy(ga, g1_v)
        pltpu.sync_copy(gb, g2_v)
        pltpu.sync_copy(pkr.at[pl.ds(ebase, ESH)], pk_v)

        def _lp(i, _):
            p16 = pk_v[pl.ds(i * 16, 16)]
            z = (plsc.load_gather(g1_v, [_unpack_src(p16)])
                 + plsc.load_gather(g2_v, [_unpack_dst(p16)]))
            out_v[pl.ds(i * 16, 16)] = _sig(z)
            return 0
        lax.fori_loop(0, ESH // 16, _lp, 0)
        pltpu.sync_copy(out_v, outr.at[pl.ds(ebase, ESH)])


_sc_scales = pl.kernel(
    _sc_scales_body,
    out_type=(jax.ShapeDtypeStruct((EPAD,), jnp.float32),
              jax.ShapeDtypeStruct((EPAD,), jnp.float32)),
    mesh=_mesh,
    compiler_params=_sc_params,
    scratch_types=[
        pltpu.VMEM((NPAD,), jnp.float32),
        pltpu.VMEM((NPAD,), jnp.float32),
        pltpu.VMEM((ESH,), jnp.int32),
        pltpu.VMEM((ESH,), jnp.float32),
    ],
)


def _zero_acc(acc, zacc_v, s):
    for z in range(12):
        pltpu.sync_copy(zacc_v, acc.at[pl.ds(s * 400 + z * 32, 32)])
    pltpu.sync_copy(zacc_v.at[pl.ds(0, 16)], acc.at[pl.ds(s * 400 + 384, 16)])


NBUF = 6
PADB = NBUF * K


def _scan_chunk(pk, sc, pe_v, sce_v, cp_v, cc_v, s, lo):
    def _scan(i, m):
        p16 = pe_v[pl.ds(i * 16, 16)]
        f16 = sce_v[pl.ds(i * 16, 16)]
        off16 = _unpack_dst(p16) - lo
        msk = (off16 >= 0) & (off16 < CH)
        plsc.store_compressed(cp_v.at[pl.ds(m, 16)], p16, mask=msk)
        plsc.store_compressed(cc_v.at[pl.ds(m, 16)], f16, mask=msk)
        return m + plsc.all_reduce_population_count(msk)[0]

    m = 0
    for h in range(2):
        base = s * STRIPE + h * HSTRIPE
        pltpu.sync_copy(pk.at[pl.ds(base, HSTRIPE)], pe_v)
        pltpu.sync_copy(sc.at[pl.ds(base, HSTRIPE)], sce_v)
        m = lax.fori_loop(0, HSTRIPE // 16, _scan, m)

    dummp = jnp.full((16,), (lo + CH + 8) << 16, jnp.int32)
    zf16 = jnp.zeros((16,), jnp.float32)
    for t in range(PADB // 16):
        cp_v[pl.ds(m + t * 16, 16)] = dummp
        cc_v[pl.ds(m + t * 16, 16)] = zf16
    ngrp = jnp.maximum((m + PADB - 1) // PADB, 1)
    return ngrp * NBUF


def _flush(tab, comp, nbat, lo, cp_v, cc_v, gidxs, sidxs, rowss,
           acc, sems_g, sems_s):
    def stage_g(j, b):
        for q in range(K // 16):
            p16 = cp_v[pl.ds(j * K + q * 16, 16)]
            gidxs[b][q * 16:(q + 1) * 16] = _unpack_src(p16)
        pltpu.async_copy(tab.at[gidxs[b]], rowss[b], sems_g[b])

    def prep_sidx(j, b):
        for q in range(K // 16):
            p16 = cp_v[pl.ds(j * K + q * 16, 16)]
            sidxs[b][q * 16:(q + 1) * 16] = _unpack_dst(p16) - lo

    def scale_rows(j, b):
        rows = rowss[b]

        @plsc.parallel_loop(0, K // 8, 1, unroll=2)
        def _rb(r8):
            sv = cc_v[pl.ds(j * K + r8 * 8, 16)]
            if comp:
                sv = 1.0 - sv
            for dr in range(8):
                scv = sv[dr]
                r = r8 * 8 + dr
                for q in range(8):
                    rows[r, q * 16:(q + 1) * 16] = (
                        scv * rows[r, q * 16:(q + 1) * 16])

    for b in range(NBUF - 2):
        stage_g(b, b)

    def _grp(t, _):
        for b in range(NBUF):
            j = t * NBUF + b
            pltpu.make_async_copy(tab.at[gidxs[b]], rowss[b],
                                  sems_g[b]).wait()
            prep_sidx(j, b)
            scale_rows(j, b)
            pltpu.async_copy(rowss[b], acc.at[sidxs[b]], sems_s[b], add=True)
            jn = j + NBUF - 2
            bn = (b + NBUF - 2) % NBUF

            @pl.when(jn < nbat)
            def _():
                @pl.when(jn >= NBUF)
                def _():
                    pltpu.make_async_copy(rowss[bn], acc.at[sidxs[bn]],
                                          sems_s[bn]).wait()
                stage_g(jn, bn)
        return 0

    lax.fori_loop(0, nbat // NBUF, _grp, 0)
    for b in range(NBUF):
        pltpu.make_async_copy(rowss[b], acc.at[sidxs[b]], sems_s[b]).wait()


def _dump(acc, out, s, lo):
    pltpu.sync_copy(acc.at[pl.ds(s * 392, 392)],
                    out.at[pl.ds(lo + s * 392, 392)])


def _sc_scatter12_body(tab1, tab2, pk, sc, out1, out2,
                       pe_v, sce_v, cp_v, cc_v,
                       g0, g1, g2, g3, g4, g5, i0, i1, i2, i3, i4, i5,
                       r0, r1, r2, r3, r4, r5, zacc_v, acc,
                       sg0, sg1, sg2, sg3, sg4, sg5,
                       ss0, ss1, ss2, ss3, ss4, ss5):
    c = lax.axis_index("c")
    s = lax.axis_index("s")
    zero16 = jnp.zeros((16,), jnp.float32)
    gidxs = (g0, g1, g2, g3, g4, g5)
    sidxs = (i0, i1, i2, i3, i4, i5)
    rowss = (r0, r1, r2, r3, r4, r5)
    sems_g = (sg0, sg1, sg2, sg3, sg4, sg5)
    sems_s = (ss0, ss1, ss2, ss3, ss4, ss5)

    def _z(i, _):
        for q in range(8):
            zacc_v[i, q * 16:(q + 1) * 16] = zero16
        return 0
    lax.fori_loop(0, 32, _z, 0)

    def _chunk(k, _):
        lo = (c * 4 + k) * CH
        _zero_acc(acc, zacc_v, s)
        plsc.subcore_barrier()
        with jax.named_scope("scan"):
            nbat = _scan_chunk(pk, sc, pe_v, sce_v, cp_v, cc_v, s, lo)
        for rep, (tab, out, comp) in enumerate(((tab1, out1, False),
                                                (tab2, out2, True))):
            with jax.named_scope("flush"):
                _flush(tab, comp, nbat, lo, cp_v, cc_v, gidxs, sidxs,
                       rowss, acc, sems_g, sems_s)
            plsc.subcore_barrier()
            with jax.named_scope("dump"):
                _dump(acc, out, s, lo)
            plsc.subcore_barrier()
            if rep == 0:
                _zero_acc(acc, zacc_v, s)
                plsc.subcore_barrier()
        return 0

    lax.fori_loop(0, 4, _chunk, 0)


def _sc_scatter3_body(tab, pk, sc, out,
                      pe_v, sce_v, cp_v, cc_v,
                      g0, g1, g2, g3, g4, g5, i0, i1, i2, i3, i4, i5,
                      r0, r1, r2, r3, r4, r5, zacc_v, acc,
                      sg0, sg1, sg2, sg3, sg4, sg5,
                      ss0, ss1, ss2, ss3, ss4, ss5):
    c = lax.axis_index("c")
    s = lax.axis_index("s")
    zero16 = jnp.zeros((16,), jnp.float32)
    gidxs = (g0, g1, g2, g3, g4, g5)
    sidxs = (i0, i1, i2, i3, i4, i5)
    rowss = (r0, r1, r2, r3, r4, r5)
    sems_g = (sg0, sg1, sg2, sg3, sg4, sg5)
    sems_s = (ss0, ss1, ss2, ss3, ss4, ss5)

    def _z(i, _):
        for q in range(8):
            zacc_v[i, q * 16:(q + 1) * 16] = zero16
        return 0
    lax.fori_loop(0, 32, _z, 0)

    def _chunk(k, _):
        lo = (c * 4 + k) * CH
        _zero_acc(acc, zacc_v, s)
        plsc.subcore_barrier()
        with jax.named_scope("scan"):
            nbat = _scan_chunk(pk, sc, pe_v, sce_v, cp_v, cc_v, s, lo)
        with jax.named_scope("flush"):
            _flush(tab, False, nbat, lo, cp_v, cc_v, gidxs, sidxs, rowss,
                   acc, sems_g, sems_s)
        plsc.subcore_barrier()
        with jax.named_scope("dump"):
            _dump(acc, out, s, lo)
        plsc.subcore_barrier()
        return 0

    lax.fori_loop(0, 4, _chunk, 0)


_scatter_scratch = (
    [pltpu.VMEM((HSTRIPE,), jnp.int32),
     pltpu.VMEM((HSTRIPE,), jnp.float32),
     pltpu.VMEM((STRIPE + 2 * PADB,), jnp.int32),
     pltpu.VMEM((STRIPE + 2 * PADB,), jnp.float32)]
    + [pltpu.VMEM((K,), jnp.int32) for _ in range(2 * NBUF)]
    + [pltpu.VMEM((K, 128), jnp.float32) for _ in range(NBUF)]
    + [pltpu.VMEM((32, 128), jnp.float32),
       pltpu.VMEM_SHARED((ACCR, 128), jnp.float32)]
    + [pltpu.SemaphoreType.DMA for _ in range(2 * NBUF)]
)

_sc_scatter12 = pl.kernel(
    _sc_scatter12_body,
    out_type=(jax.ShapeDtypeStruct((NPAD, 128), jnp.float32),
              jax.ShapeDtypeStruct((NPAD, 128), jnp.float32)),
    mesh=_mesh,
    compiler_params=_sc_params,
    scratch_types=_scatter_scratch,
)

_sc_scatter3 = pl.kernel(
    _sc_scatter3_body,
    out_type=jax.ShapeDtypeStruct((NPAD, 128), jnp.float32),
    mesh=_mesh,
    compiler_params=_sc_params,
    scratch_types=_scatter_scratch,
)


BR = 3136


def _mm_bias_kern(x_ref, w_ref, b_ref, o_ref):
    o_ref[...] = (jnp.dot(x_ref[...], w_ref[...],
                          preferred_element_type=jnp.float32)
                  + b_ref[...])


def _mm_bias(x, w, b):
    p = w.shape[1]
    return pl.pallas_call(
        _mm_bias_kern,
        grid=(NPAD // BR,),
        in_specs=[
            pl.BlockSpec((BR, 128), lambda i: (i, 0)),
            pl.BlockSpec((128, p), lambda i: (0, 0)),
            pl.BlockSpec((1, p), lambda i: (0, 0)),
        ],
        out_specs=pl.BlockSpec((BR, p), lambda i: (i, 0)),
        out_shape=jax.ShapeDtypeStruct((NPAD, p), jnp.float32),
    )(x, w, b.reshape(1, p))


def _combine_kern(s1_ref, s2_ref, s3_ref, cnt_ref, emb_ref, o_ref):
    rc = 1.0 / jnp.maximum(cnt_ref[...], 1.0)
    o_ref[...] = 0.5 * (jnp.maximum(s1_ref[...] * rc, 0.0)
                        + jnp.maximum(s2_ref[...] * s3_ref[...], 0.0)
                        + emb_ref[...])


def _combine(s1, s2, s3, cnt, emb):
    return pl.pallas_call(
        _combine_kern,
        grid=(NPAD // BR,),
        in_specs=[
            pl.BlockSpec((BR, 128), lambda i: (i, 0)),
            pl.BlockSpec((BR, 128), lambda i: (i, 0)),
            pl.BlockSpec((BR, 128), lambda i: (i, 0)),
            pl.BlockSpec((BR, 1), lambda i: (i, 0)),
            pl.BlockSpec((BR, 128), lambda i: (i, 0)),
        ],
        out_specs=pl.BlockSpec((BR, 128), lambda i: (i, 0)),
        out_shape=jax.ShapeDtypeStruct((NPAD, 128), jnp.float32),
    )(s1, s2, s3, cnt.reshape(NPAD, 1), emb)


def kernel(x, params, b0_cor, b0_sim, b1_cor, b1_sim):
    xpad = jnp.pad(x, ((0, NPAD - N), (0, 0)))
    names = ("b0_cor", "b0_sim", "b1_cor", "b1_sim")
    arrs = (b0_cor, b0_sim, b1_cor, b1_sim)
    epad = {nm: jnp.pad(a, ((0, 0), (0, EPAD - E)), constant_values=NPAD - 1)
            for nm, a in zip(names, arrs)}

    src_all = jnp.concatenate([epad[nm][0] for nm in names])
    dst_all = jnp.concatenate([epad[nm][1] for nm in names])
    cnt_flat, pk_flat = _sc_prep(src_all, dst_all)
    cnt = cnt_flat.reshape(4, NPAD)
    pk = {nm: pk_flat[i * EPAD:(i + 1) * EPAD] for i, nm in enumerate(names)}

    cnt_row = {("cor", 0): 0, ("sim", 0): 1, ("cor", 1): 2, ("sim", 1): 3}
    blocks = [("b0_cor", "b0_sim"), ("b1_cor", "b1_sim")]

    outs = []
    for mode in ("cor", "sim"):
        pm = params[mode]
        emb = _mm_bias(xpad, pm["fcin_w"], pm["fcin_b"])
        for li, (cor_nm, sim_nm) in enumerate(blocks):
            em_nm, eo_nm = ((cor_nm, sim_nm) if mode == "cor"
                            else (sim_nm, cor_nm))
            patt = pm["l1_att"] if li == 0 else pm["l2_att"]
            pagg = pm["l1_agg"] if li == 0 else pm["l2_agg"]
            ea, ia = patt["e_att"], patt["i_att"]
            ds_w = (ea[:D, 0] - ea[:D, 1])[:, None]
            dd_w = (ea[D:, 0] - ea[D:, 1])[:, None]
            wcat = jnp.concatenate(
                [pagg["W"], ds_w, dd_w, ia[:D, 0:1], ia[D:, 0:1],
                 jnp.zeros((D, 124), jnp.float32)], axis=1)
            bcat = jnp.concatenate([pagg["b"], jnp.zeros((128,), jnp.float32)])
            y = _mm_bias(emb, wcat, bcat)
            h = y[:, :D]
            gs, gd = y[:, D], y[:, D + 1]
            t1, t2 = y[:, D + 2], y[:, D + 3]

            scm, sco = _sc_scales(pk[em_nm], pk[eo_nm], gs, gd, t1, t2)
            s1, s2 = _sc_scatter12(h, emb, pk[em_nm], scm)
            s3 = _sc_scatter3(emb, pk[eo_nm], sco)
            emb = _combine(s1, s2, s3, cnt[cnt_row[(mode, li)]], emb)
        outs.append(_mm_bias(emb, pm["fcout_w"], pm["fcout_b"])[:N])
    return tuple(outs)


# speedup vs baseline: 1.9112x; 1.9112x over previous
import jax
import jax.numpy as jnp
from jax import lax
from jax.experimental import pallas as pl
from jax.experimental.pallas import tpu as pltpu
from jax.experimental.pallas import tpu_sc as plsc

N = 50000
D = 128
E = 200000

NPAD = 50176
EPAD = 200192
CH = 6272
ACCR = 6400
STRIPE = EPAD // 16
HSTRIPE = STRIPE // 2
ESH = EPAD // 32
K = 32
PAIR = 2 * K

_mesh = plsc.VectorSubcoreMesh(core_axis_name="c", subcore_axis_name="s")
_sc_params = pltpu.CompilerParams(needs_layout_passes=False)


def _sig(z):
    return 1.0 / (1.0 + jnp.exp(-z))


def _unpack_src(p16):
    return p16 & 0xFFFF


def _unpack_dst(p16):
    return lax.shift_right_logical(p16, 16)


def _sc_prep_body(src_all, dst_all, cnt, pk, acc_l, se_v, de_v, pk_v, tmp,
                  res, stage):
    c = lax.axis_index("c")
    s = lax.axis_index("s")
    zero16 = jnp.zeros((16,), jnp.float32)
    ones16 = jnp.ones((16,), jnp.float32)

    for a in range(2):
        arr = c * 2 + a

        def _z(i, _):
            for q in range(8):
                acc_l[pl.ds((i * 8 + q) * 16, 16)] = zero16
            return 0
        lax.fori_loop(0, NPAD // 128, _z, 0)

        def _scan(i, _):
            s16 = se_v[pl.ds(i * 16, 16)]
            d16 = de_v[pl.ds(i * 16, 16)]
            pk_v[pl.ds(i * 16, 16)] = (d16 << 16) | s16
            plsc.addupdate_scatter(acc_l, [d16], ones16)
            return 0

        for h in range(2):
            base = arr * EPAD + s * STRIPE + h * HSTRIPE
            pltpu.sync_copy(src_all.at[pl.ds(base, HSTRIPE)], se_v)
            pltpu.sync_copy(dst_all.at[pl.ds(base, HSTRIPE)], de_v)
            lax.fori_loop(0, HSTRIPE // 16, _scan, 0)
            pltpu.sync_copy(pk_v, pk.at[pl.ds(base, HSTRIPE)])

        pltpu.sync_copy(acc_l, stage.at[pl.ds(s * NPAD, NPAD)])
        plsc.subcore_barrier()

        colbase = s * (NPAD // 16)
        pltpu.sync_copy(stage.at[pl.ds(colbase, NPAD // 16)], res)
        for j in range(1, 16):
            pltpu.sync_copy(stage.at[pl.ds(j * NPAD + colbase, NPAD // 16)],
                            tmp)

            def _add(i, _):
                for q in range(4):
                    o = (i * 4 + q) * 16
                    res[pl.ds(o, 16)] = (res[pl.ds(o, 16)]
                                         + tmp[pl.ds(o, 16)])
                return 0
            lax.fori_loop(0, NPAD // 16 // 64, _add, 0)
        pltpu.sync_copy(res, cnt.at[pl.ds(arr * NPAD + colbase, NPAD // 16)])
        plsc.subcore_barrier()


_sc_prep = pl.kernel(
    _sc_prep_body,
    out_type=(jax.ShapeDtypeStruct((4 * NPAD,), jnp.float32),
              jax.ShapeDtypeStruct((4 * EPAD,), jnp.int32)),
    mesh=_mesh,
    compiler_params=_sc_params,
    scratch_types=[
        pltpu.VMEM((NPAD,), jnp.float32),
        pltpu.VMEM((HSTRIPE,), jnp.int32),
        pltpu.VMEM((HSTRIPE,), jnp.int32),
        pltpu.VMEM((HSTRIPE,), jnp.int32),
        pltpu.VMEM((NPAD // 16,), jnp.float32),
        pltpu.VMEM((NPAD // 16,), jnp.float32),
        pltpu.VMEM_SHARED((16 * NPAD,), jnp.float32),
    ],
)


def _sc_scales_body(pk_em, pk_eo, gs, gd, t1, t2, scm, sco,
                    g1_v, g2_v, pk_v, out_v):
    c = lax.axis_index("c")
    s = lax.axis_index("s")
    wid = s * 2 + c
    ebase = wid * ESH

    for part, (ga, gb, pkr, outr) in enumerate(
            (((gs, gd, pk_em, scm)), (t1, t2, pk_eo, sco))):
        pltpu.sync_copy(ga, g1_v)
        pltpu.sync_copy(gb, g2_v)
        pltpu.sync_copy(pkr.at[pl.ds(ebase, ESH)], pk_v)

        def _lp(i, _):
            p16 = pk_v[pl.ds(i * 16, 16)]
            z = (plsc.load_gather(g1_v, [_unpack_src(p16)])
                 + plsc.load_gather(g2_v, [_unpack_dst(p16)]))
            out_v[pl.ds(i * 16, 16)] = _sig(z)
            return 0
        lax.fori_loop(0, ESH // 16, _lp, 0)
        pltpu.sync_copy(out_v, outr.at[pl.ds(ebase, ESH)])


_sc_scales = pl.kernel(
    _sc_scales_body,
    out_type=(jax.ShapeDtypeStruct((EPAD,), jnp.float32),
              jax.ShapeDtypeStruct((EPAD,), jnp.float32)),
    mesh=_mesh,
    compiler_params=_sc_params,
    scratch_types=[
        pltpu.VMEM((NPAD,), jnp.float32),
        pltpu.VMEM((NPAD,), jnp.float32),
        pltpu.VMEM((ESH,), jnp.int32),
        pltpu.VMEM((ESH,), jnp.float32),
    ],
)


def _zero_acc(acc, zacc_v, s):
    for z in range(12):
        pltpu.sync_copy(zacc_v, acc.at[pl.ds(s * 400 + z * 32, 32)])
    pltpu.sync_copy(zacc_v.at[pl.ds(0, 16)], acc.at[pl.ds(s * 400 + 384, 16)])


def _scan_chunk(pk, sc, pe_v, sce_v, cp_v, cc_v, s, lo):
    def _scan(i, m):
        p16 = pe_v[pl.ds(i * 16, 16)]
        f16 = sce_v[pl.ds(i * 16, 16)]
        off16 = _unpack_dst(p16) - lo
        msk = (off16 >= 0) & (off16 < CH)
        plsc.store_compressed(cp_v.at[pl.ds(m, 16)], p16, mask=msk)
        plsc.store_compressed(cc_v.at[pl.ds(m, 16)], f16, mask=msk)
        return m + jnp.sum(msk.astype(jnp.int32))

    m = 0
    for h in range(2):
        base = s * STRIPE + h * HSTRIPE
        pltpu.sync_copy(pk.at[pl.ds(base, HSTRIPE)], pe_v)
        pltpu.sync_copy(sc.at[pl.ds(base, HSTRIPE)], sce_v)
        m = lax.fori_loop(0, HSTRIPE // 16, _scan, m)

    dummp = jnp.full((16,), (lo + CH + 8) << 16, jnp.int32)
    zf16 = jnp.zeros((16,), jnp.float32)
    for t in range(PAIR // 16):
        cp_v[pl.ds(m + t * 16, 16)] = dummp
        cc_v[pl.ds(m + t * 16, 16)] = zf16
    return jnp.maximum((m + PAIR - 1) // PAIR, 1)


def _flush(tab, comp, npairs, lo, cp_v, cc_v, gidx0, gidx1, sidx0, sidx1,
           rows0, rows1, acc, sem_g0, sem_g1, sem_s0, sem_s1):
    def stage_g(j, gidx, rows, sem):
        for q in range(K // 16):
            p16 = cp_v[pl.ds(j * K + q * 16, 16)]
            gidx[q * 16:(q + 1) * 16] = _unpack_src(p16)
        pltpu.async_copy(tab.at[gidx], rows, sem)

    def prep_sidx(j, sidx):
        for q in range(K // 16):
            p16 = cp_v[pl.ds(j * K + q * 16, 16)]
            sidx[q * 16:(q + 1) * 16] = _unpack_dst(p16) - lo

    def scale_rows(j, rows):
        @plsc.parallel_loop(0, K // 8, 1, unroll=2)
        def _rb(r8):
            sv = cc_v[pl.ds(j * K + r8 * 8, 16)]
            if comp:
                sv = 1.0 - sv
            for dr in range(8):
                scv = sv[dr]
                r = r8 * 8 + dr
                for q in range(8):
                    rows[r, q * 16:(q + 1) * 16] = (
                        scv * rows[r, q * 16:(q + 1) * 16])

    stage_g(0, gidx0, rows0, sem_g0)

    def _pair(t, _):
        j0 = 2 * t

        @pl.when(t > 0)
        def _():
            pltpu.make_async_copy(rows1, acc.at[sidx1], sem_s1).wait()
        stage_g(j0 + 1, gidx1, rows1, sem_g1)
        pltpu.make_async_copy(tab.at[gidx0], rows0, sem_g0).wait()
        prep_sidx(j0, sidx0)
        scale_rows(j0, rows0)
        pltpu.async_copy(rows0, acc.at[sidx0], sem_s0, add=True)

        pltpu.make_async_copy(tab.at[gidx1], rows1, sem_g1).wait()
        prep_sidx(j0 + 1, sidx1)
        scale_rows(j0 + 1, rows1)

        @pl.when(t + 1 < npairs)
        def _():
            pltpu.make_async_copy(rows0, acc.at[sidx0], sem_s0).wait()
            stage_g(j0 + 2, gidx0, rows0, sem_g0)
        pltpu.async_copy(rows1, acc.at[sidx1], sem_s1, add=True)
        return 0

    lax.fori_loop(0, npairs, _pair, 0)
    pltpu.make_async_copy(rows0, acc.at[sidx0], sem_s0).wait()
    pltpu.make_async_copy(rows1, acc.at[sidx1], sem_s1).wait()


def _dump(acc, out, s, lo):
    pltpu.sync_copy(acc.at[pl.ds(s * 392, 392)],
                    out.at[pl.ds(lo + s * 392, 392)])


def _sc_scatter12_body(tab1, tab2, pk, sc, out1, out2,
                       pe_v, sce_v, cp_v, cc_v, gidx0, gidx1, sidx0, sidx1,
                       rows0, rows1, zacc_v, acc,
                       sem_g0, sem_g1, sem_s0, sem_s1):
    c = lax.axis_index("c")
    s = lax.axis_index("s")
    zero16 = jnp.zeros((16,), jnp.float32)

    def _z(i, _):
        for q in range(8):
            zacc_v[i, q * 16:(q + 1) * 16] = zero16
        return 0
    lax.fori_loop(0, 32, _z, 0)

    bufs = (cp_v, cc_v, gidx0, gidx1, sidx0, sidx1, rows0, rows1, acc,
            sem_g0, sem_g1, sem_s0, sem_s1)
    for k in range(4):
        lo = (c * 4 + k) * CH
        _zero_acc(acc, zacc_v, s)
        plsc.subcore_barrier()
        nbat = _scan_chunk(pk, sc, pe_v, sce_v, cp_v, cc_v, s, lo)
        for rep, (tab, out, comp) in enumerate(((tab1, out1, False),
                                                (tab2, out2, True))):
            _flush(tab, comp, nbat, lo, *bufs)
            plsc.subcore_barrier()
            _dump(acc, out, s, lo)
            plsc.subcore_barrier()
            if rep == 0:
                _zero_acc(acc, zacc_v, s)
                plsc.subcore_barrier()


def _sc_scatter3_body(tab, pk, sc, out,
                      pe_v, sce_v, cp_v, cc_v, gidx0, gidx1, sidx0, sidx1,
                      rows0, rows1, zacc_v, acc,
                      sem_g0, sem_g1, sem_s0, sem_s1):
    c = lax.axis_index("c")
    s = lax.axis_index("s")
    zero16 = jnp.zeros((16,), jnp.float32)

    def _z(i, _):
        for q in range(8):
            zacc_v[i, q * 16:(q + 1) * 16] = zero16
        return 0
    lax.fori_loop(0, 32, _z, 0)

    for k in range(4):
        lo = (c * 4 + k) * CH
        _zero_acc(acc, zacc_v, s)
        plsc.subcore_barrier()
        nbat = _scan_chunk(pk, sc, pe_v, sce_v, cp_v, cc_v, s, lo)
        _flush(tab, False, nbat, lo, cp_v, cc_v, gidx0, gidx1, sidx0,
               sidx1, rows0, rows1, acc, sem_g0, sem_g1, sem_s0, sem_s1)
        plsc.subcore_barrier()
        _dump(acc, out, s, lo)
        plsc.subcore_barrier()


_scatter_scratch = [
    pltpu.VMEM((HSTRIPE,), jnp.int32),
    pltpu.VMEM((HSTRIPE,), jnp.float32),
    pltpu.VMEM((STRIPE + 2 * PAIR,), jnp.int32),
    pltpu.VMEM((STRIPE + 2 * PAIR,), jnp.float32),
    pltpu.VMEM((K,), jnp.int32),
    pltpu.VMEM((K,), jnp.int32),
    pltpu.VMEM((K,), jnp.int32),
    pltpu.VMEM((K,), jnp.int32),
    pltpu.VMEM((K, 128), jnp.float32),
    pltpu.VMEM((K, 128), jnp.float32),
    pltpu.VMEM((32, 128), jnp.float32),
    pltpu.VMEM_SHARED((ACCR, 128), jnp.float32),
    pltpu.SemaphoreType.DMA,
    pltpu.SemaphoreType.DMA,
    pltpu.SemaphoreType.DMA,
    pltpu.SemaphoreType.DMA,
]

_sc_scatter12 = pl.kernel(
    _sc_scatter12_body,
    out_type=(jax.ShapeDtypeStruct((NPAD, 128), jnp.float32),
              jax.ShapeDtypeStruct((NPAD, 128), jnp.float32)),
    mesh=_mesh,
    compiler_params=_sc_params,
    scratch_types=_scatter_scratch,
)

_sc_scatter3 = pl.kernel(
    _sc_scatter3_body,
    out_type=jax.ShapeDtypeStruct((NPAD, 128), jnp.float32),
    mesh=_mesh,
    compiler_params=_sc_params,
    scratch_types=_scatter_scratch,
)


BR = 3136


def _mm_bias_kern(x_ref, w_ref, b_ref, o_ref):
    o_ref[...] = (jnp.dot(x_ref[...], w_ref[...],
                          preferred_element_type=jnp.float32)
                  + b_ref[...])


def _mm_bias(x, w, b):
    p = w.shape[1]
    return pl.pallas_call(
        _mm_bias_kern,
        grid=(NPAD // BR,),
        in_specs=[
            pl.BlockSpec((BR, 128), lambda i: (i, 0)),
            pl.BlockSpec((128, p), lambda i: (0, 0)),
            pl.BlockSpec((1, p), lambda i: (0, 0)),
        ],
        out_specs=pl.BlockSpec((BR, p), lambda i: (i, 0)),
        out_shape=jax.ShapeDtypeStruct((NPAD, p), jnp.float32),
    )(x, w, b.reshape(1, p))


def _combine_kern(s1_ref, s2_ref, s3_ref, cnt_ref, emb_ref, o_ref):
    rc = 1.0 / jnp.maximum(cnt_ref[...], 1.0)
    o_ref[...] = 0.5 * (jnp.maximum(s1_ref[...] * rc, 0.0)
                        + jnp.maximum(s2_ref[...] * s3_ref[...], 0.0)
                        + emb_ref[...])


def _combine(s1, s2, s3, cnt, emb):
    return pl.pallas_call(
        _combine_kern,
        grid=(NPAD // BR,),
        in_specs=[
            pl.BlockSpec((BR, 128), lambda i: (i, 0)),
            pl.BlockSpec((BR, 128), lambda i: (i, 0)),
            pl.BlockSpec((BR, 128), lambda i: (i, 0)),
            pl.BlockSpec((BR, 1), lambda i: (i, 0)),
            pl.BlockSpec((BR, 128), lambda i: (i, 0)),
        ],
        out_specs=pl.BlockSpec((BR, 128), lambda i: (i, 0)),
        out_shape=jax.ShapeDtypeStruct((NPAD, 128), jnp.float32),
    )(s1, s2, s3, cnt.reshape(NPAD, 1), emb)


def kernel(x, params, b0_cor, b0_sim, b1_cor, b1_sim):
    xpad = jnp.pad(x, ((0, NPAD - N), (0, 0)))
    names = ("b0_cor", "b0_sim", "b1_cor", "b1_sim")
    arrs = (b0_cor, b0_sim, b1_cor, b1_sim)
    epad = {nm: jnp.pad(a, ((0, 0), (0, EPAD - E)), constant_values=NPAD - 1)
            for nm, a in zip(names, arrs)}

    src_all = jnp.concatenate([epad[nm][0] for nm in names])
    dst_all = jnp.concatenate([epad[nm][1] for nm in names])
    cnt_flat, pk_flat = _sc_prep(src_all, dst_all)
    cnt = cnt_flat.reshape(4, NPAD)
    pk = {nm: pk_flat[i * EPAD:(i + 1) * EPAD] for i, nm in enumerate(names)}

    cnt_row = {("cor", 0): 0, ("sim", 0): 1, ("cor", 1): 2, ("sim", 1): 3}
    blocks = [("b0_cor", "b0_sim"), ("b1_cor", "b1_sim")]

    outs = []
    for mode in ("cor", "sim"):
        pm = params[mode]
        emb = _mm_bias(xpad, pm["fcin_w"], pm["fcin_b"])
        for li, (cor_nm, sim_nm) in enumerate(blocks):
            em_nm, eo_nm = ((cor_nm, sim_nm) if mode == "cor"
                            else (sim_nm, cor_nm))
            patt = pm["l1_att"] if li == 0 else pm["l2_att"]
            pagg = pm["l1_agg"] if li == 0 else pm["l2_agg"]
            ea, ia = patt["e_att"], patt["i_att"]
            ds_w = (ea[:D, 0] - ea[:D, 1])[:, None]
            dd_w = (ea[D:, 0] - ea[D:, 1])[:, None]
            wcat = jnp.concatenate(
                [pagg["W"], ds_w, dd_w, ia[:D, 0:1], ia[D:, 0:1],
                 jnp.zeros((D, 124), jnp.float32)], axis=1)
            bcat = jnp.concatenate([pagg["b"], jnp.zeros((128,), jnp.float32)])
            y = _mm_bias(emb, wcat, bcat)
            h = y[:, :D]
            gs, gd = y[:, D], y[:, D + 1]
            t1, t2 = y[:, D + 2], y[:, D + 3]

            scm, sco = _sc_scales(pk[em_nm], pk[eo_nm], gs, gd, t1, t2)
            s1, s2 = _sc_scatter12(h, emb, pk[em_nm], scm)
            s3 = _sc_scatter3(emb, pk[eo_nm], sco)
            emb = _combine(s1, s2, s3, cnt[cnt_row[(mode, li)]], emb)
        outs.append(_mm_bias(emb, pm["fcout_w"], pm["fcout_b"])[:N])
    return tuple(outs)


# speedup vs baseline: 2.0468x; 1.0709x over previous
import jax
import jax.numpy as jnp
from jax import lax
from jax.experimental import pallas as pl
from jax.experimental.pallas import tpu as pltpu
from jax.experimental.pallas import tpu_sc as plsc

N = 50000
D = 128
E = 200000

NPAD = 50176
EPAD = 200192
CH = 6272
ACCR = 6400
STRIPE = EPAD // 16
HSTRIPE = STRIPE // 2
ESH = EPAD // 32
K = 16
PAIR = 2 * K

_mesh = plsc.VectorSubcoreMesh(core_axis_name="c", subcore_axis_name="s")
_sc_params = pltpu.CompilerParams(needs_layout_passes=False)


def _sig(z):
    return 1.0 / (1.0 + jnp.exp(-z))


def _unpack_src(p16):
    return p16 & 0xFFFF


def _unpack_dst(p16):
    return lax.shift_right_logical(p16, 16)


def _sc_prep_body(src_all, dst_all, cnt, pk, acc_l, se_v, de_v, pk_v, tmp,
                  res, stage):
    c = lax.axis_index("c")
    s = lax.axis_index("s")
    zero16 = jnp.zeros((16,), jnp.float32)
    ones16 = jnp.ones((16,), jnp.float32)

    for a in range(2):
        arr = c * 2 + a

        def _z(i, _):
            for q in range(8):
                acc_l[pl.ds((i * 8 + q) * 16, 16)] = zero16
            return 0
        lax.fori_loop(0, NPAD // 128, _z, 0)

        def _scan(i, _):
            s16 = se_v[pl.ds(i * 16, 16)]
            d16 = de_v[pl.ds(i * 16, 16)]
            pk_v[pl.ds(i * 16, 16)] = (d16 << 16) | s16
            plsc.addupdate_scatter(acc_l, [d16], ones16)
            return 0

        for h in range(2):
            base = arr * EPAD + s * STRIPE + h * HSTRIPE
            pltpu.sync_copy(src_all.at[pl.ds(base, HSTRIPE)], se_v)
            pltpu.sync_copy(dst_all.at[pl.ds(base, HSTRIPE)], de_v)
            lax.fori_loop(0, HSTRIPE // 16, _scan, 0)
            pltpu.sync_copy(pk_v, pk.at[pl.ds(base, HSTRIPE)])

        pltpu.sync_copy(acc_l, stage.at[pl.ds(s * NPAD, NPAD)])
        plsc.subcore_barrier()

        colbase = s * (NPAD // 16)
        pltpu.sync_copy(stage.at[pl.ds(colbase, NPAD // 16)], res)
        for j in range(1, 16):
            pltpu.sync_copy(stage.at[pl.ds(j * NPAD + colbase, NPAD // 16)],
                            tmp)

            def _add(i, _):
                for q in range(4):
                    o = (i * 4 + q) * 16
                    res[pl.ds(o, 16)] = (res[pl.ds(o, 16)]
                                         + tmp[pl.ds(o, 16)])
                return 0
            lax.fori_loop(0, NPAD // 16 // 64, _add, 0)
        pltpu.sync_copy(res, cnt.at[pl.ds(arr * NPAD + colbase, NPAD // 16)])
        plsc.subcore_barrier()


_sc_prep = pl.kernel(
    _sc_prep_body,
    out_type=(jax.ShapeDtypeStruct((4 * NPAD,), jnp.float32),
              jax.ShapeDtypeStruct((4 * EPAD,), jnp.int32)),
    mesh=_mesh,
    compiler_params=_sc_params,
    scratch_types=[
        pltpu.VMEM((NPAD,), jnp.float32),
        pltpu.VMEM((HSTRIPE,), jnp.int32),
        pltpu.VMEM((HSTRIPE,), jnp.int32),
        pltpu.VMEM((HSTRIPE,), jnp.int32),
        pltpu.VMEM((NPAD // 16,), jnp.float32),
        pltpu.VMEM((NPAD // 16,), jnp.float32),
        pltpu.VMEM_SHARED((16 * NPAD,), jnp.float32),
    ],
)


def _sc_scales_body(pk_em, pk_eo, gs, gd, t1, t2, scm, sco,
                    g1_v, g2_v, pk_v, out_v):
    c = lax.axis_index("c")
    s = lax.axis_index("s")
    wid = s * 2 + c
    ebase = wid * ESH

    for part, (ga, gb, pkr, outr) in enumerate(
            (((gs, gd, pk_em, scm)), (t1, t2, pk_eo, sco))):
        pltpu.sync_copy(ga, g1_v)
        pltpu.sync_copy(gb, g2_v)
        pltpu.sync_copy(pkr.at[pl.ds(ebase, ESH)], pk_v)

        def _lp(i, _):
            p16 = pk_v[pl.ds(i * 16, 16)]
            z = (plsc.load_gather(g1_v, [_unpack_src(p16)])
                 + plsc.load_gather(g2_v, [_unpack_dst(p16)]))
            out_v[pl.ds(i * 16, 16)] = _sig(z)
            return 0
        lax.fori_loop(0, ESH // 16, _lp, 0)
        pltpu.sync_copy(out_v, outr.at[pl.ds(ebase, ESH)])


_sc_scales = pl.kernel(
    _sc_scales_body,
    out_type=(jax.ShapeDtypeStruct((EPAD,), jnp.float32),
              jax.ShapeDtypeStruct((EPAD,), jnp.float32)),
    mesh=_mesh,
    compiler_params=_sc_params,
    scratch_types=[
        pltpu.VMEM((NPAD,), jnp.float32),
        pltpu.VMEM((NPAD,), jnp.float32),
        pltpu.VMEM((ESH,), jnp.int32),
        pltpu.VMEM((ESH,), jnp.float32),
    ],
)


def _zero_acc(acc, zacc_v, s):
    for z in range(12):
        pltpu.sync_copy(zacc_v, acc.at[pl.ds(s * 400 + z * 32, 32)])
    pltpu.sync_copy(zacc_v.at[pl.ds(0, 16)], acc.at[pl.ds(s * 400 + 384, 16)])


def _scan_chunk(pk, sc, pe_v, sce_v, cp_v, cc_v, s, lo):
    def _scan(i, m):
        p16 = pe_v[pl.ds(i * 16, 16)]
        f16 = sce_v[pl.ds(i * 16, 16)]
        off16 = _unpack_dst(p16) - lo
        msk = (off16 >= 0) & (off16 < CH)
        plsc.store_compressed(cp_v.at[pl.ds(m, 16)], p16, mask=msk)
        plsc.store_compressed(cc_v.at[pl.ds(m, 16)], f16, mask=msk)
        return m + jnp.sum(msk.astype(jnp.int32))

    m = 0
    for h in range(2):
        base = s * STRIPE + h * HSTRIPE
        pltpu.sync_copy(pk.at[pl.ds(base, HSTRIPE)], pe_v)
        pltpu.sync_copy(sc.at[pl.ds(base, HSTRIPE)], sce_v)
        m = lax.fori_loop(0, HSTRIPE // 16, _scan, m)

    dummp = jnp.full((16,), (lo + CH + 8) << 16, jnp.int32)
    zf16 = jnp.zeros((16,), jnp.float32)
    for t in range(PAIR // 16):
        cp_v[pl.ds(m + t * 16, 16)] = dummp
        cc_v[pl.ds(m + t * 16, 16)] = zf16
    return jnp.maximum((m + PAIR - 1) // PAIR, 1)


def _flush(tab, comp, npairs, lo, cp_v, cc_v, gidx0, gidx1, sidx0, sidx1,
           rows0, rows1, acc, sem_g0, sem_g1, sem_s0, sem_s1):
    def stage_g(j, gidx, rows, sem):
        for q in range(K // 16):
            p16 = cp_v[pl.ds(j * K + q * 16, 16)]
            gidx[q * 16:(q + 1) * 16] = _unpack_src(p16)
        pltpu.async_copy(tab.at[gidx], rows, sem)

    def prep_sidx(j, sidx):
        for q in range(K // 16):
            p16 = cp_v[pl.ds(j * K + q * 16, 16)]
            sidx[q * 16:(q + 1) * 16] = _unpack_dst(p16) - lo

    def scale_rows(j, rows):
        @plsc.parallel_loop(0, K // 8, 1, unroll=2)
        def _rb(r8):
            sv = cc_v[pl.ds(j * K + r8 * 8, 16)]
            if comp:
                sv = 1.0 - sv
            for dr in range(8):
                scv = sv[dr]
                r = r8 * 8 + dr
                for q in range(8):
                    rows[r, q * 16:(q + 1) * 16] = (
                        scv * rows[r, q * 16:(q + 1) * 16])

    stage_g(0, gidx0, rows0, sem_g0)

    def _pair(t, _):
        j0 = 2 * t

        @pl.when(t > 0)
        def _():
            pltpu.make_async_copy(rows1, acc.at[sidx1], sem_s1).wait()
        stage_g(j0 + 1, gidx1, rows1, sem_g1)
        pltpu.make_async_copy(tab.at[gidx0], rows0, sem_g0).wait()
        prep_sidx(j0, sidx0)
        scale_rows(j0, rows0)
        pltpu.async_copy(rows0, acc.at[sidx0], sem_s0, add=True)

        pltpu.make_async_copy(tab.at[gidx1], rows1, sem_g1).wait()
        prep_sidx(j0 + 1, sidx1)
        scale_rows(j0 + 1, rows1)

        @pl.when(t + 1 < npairs)
        def _():
            pltpu.make_async_copy(rows0, acc.at[sidx0], sem_s0).wait()
            stage_g(j0 + 2, gidx0, rows0, sem_g0)
        pltpu.async_copy(rows1, acc.at[sidx1], sem_s1, add=True)
        return 0

    lax.fori_loop(0, npairs, _pair, 0)
    pltpu.make_async_copy(rows0, acc.at[sidx0], sem_s0).wait()
    pltpu.make_async_copy(rows1, acc.at[sidx1], sem_s1).wait()


def _dump(acc, out, s, lo):
    pltpu.sync_copy(acc.at[pl.ds(s * 392, 392)],
                    out.at[pl.ds(lo + s * 392, 392)])


def _sc_scatter12_body(tab1, tab2, pk, sc, out1, out2,
                       pe_v, sce_v, cp_v, cc_v, gidx0, gidx1, sidx0, sidx1,
                       rows0, rows1, zacc_v, acc,
                       sem_g0, sem_g1, sem_s0, sem_s1):
    c = lax.axis_index("c")
    s = lax.axis_index("s")
    zero16 = jnp.zeros((16,), jnp.float32)

    def _z(i, _):
        for q in range(8):
            zacc_v[i, q * 16:(q + 1) * 16] = zero16
        return 0
    lax.fori_loop(0, 32, _z, 0)

    bufs = (cp_v, cc_v, gidx0, gidx1, sidx0, sidx1, rows0, rows1, acc,
            sem_g0, sem_g1, sem_s0, sem_s1)
    for k in range(4):
        lo = (c * 4 + k) * CH
        _zero_acc(acc, zacc_v, s)
        plsc.subcore_barrier()
        nbat = _scan_chunk(pk, sc, pe_v, sce_v, cp_v, cc_v, s, lo)
        for rep, (tab, out, comp) in enumerate(((tab1, out1, False),
                                                (tab2, out2, True))):
            _flush(tab, comp, nbat, lo, *bufs)
            plsc.subcore_barrier()
            _dump(acc, out, s, lo)
            plsc.subcore_barrier()
            if rep == 0:
                _zero_acc(acc, zacc_v, s)
                plsc.subcore_barrier()


def _sc_scatter3_body(tab, pk, sc, out,
                      pe_v, sce_v, cp_v, cc_v, gidx0, gidx1, sidx0, sidx1,
                      rows0, rows1, zacc_v, acc,
                      sem_g0, sem_g1, sem_s0, sem_s1):
    c = lax.axis_index("c")
    s = lax.axis_index("s")
    zero16 = jnp.zeros((16,), jnp.float32)

    def _z(i, _):
        for q in range(8):
            zacc_v[i, q * 16:(q + 1) * 16] = zero16
        return 0
    lax.fori_loop(0, 32, _z, 0)

    for k in range(4):
        lo = (c * 4 + k) * CH
        _zero_acc(acc, zacc_v, s)
        plsc.subcore_barrier()
        nbat = _scan_chunk(pk, sc, pe_v, sce_v, cp_v, cc_v, s, lo)
        _flush(tab, False, nbat, lo, cp_v, cc_v, gidx0, gidx1, sidx0,
               sidx1, rows0, rows1, acc, sem_g0, sem_g1, sem_s0, sem_s1)
        plsc.subcore_barrier()
        _dump(acc, out, s, lo)
        plsc.subcore_barrier()


_scatter_scratch = [
    pltpu.VMEM((HSTRIPE,), jnp.int32),
    pltpu.VMEM((HSTRIPE,), jnp.float32),
    pltpu.VMEM((STRIPE + 2 * PAIR,), jnp.int32),
    pltpu.VMEM((STRIPE + 2 * PAIR,), jnp.float32),
    pltpu.VMEM((K,), jnp.int32),
    pltpu.VMEM((K,), jnp.int32),
    pltpu.VMEM((K,), jnp.int32),
    pltpu.VMEM((K,), jnp.int32),
    pltpu.VMEM((K, 128), jnp.float32),
    pltpu.VMEM((K, 128), jnp.float32),
    pltpu.VMEM((32, 128), jnp.float32),
    pltpu.VMEM_SHARED((ACCR, 128), jnp.float32),
    pltpu.SemaphoreType.DMA,
    pltpu.SemaphoreType.DMA,
    pltpu.SemaphoreType.DMA,
    pltpu.SemaphoreType.DMA,
]

_sc_scatter12 = pl.kernel(
    _sc_scatter12_body,
    out_type=(jax.ShapeDtypeStruct((NPAD, 128), jnp.float32),
              jax.ShapeDtypeStruct((NPAD, 128), jnp.float32)),
    mesh=_mesh,
    compiler_params=_sc_params,
    scratch_types=_scatter_scratch,
)

_sc_scatter3 = pl.kernel(
    _sc_scatter3_body,
    out_type=jax.ShapeDtypeStruct((NPAD, 128), jnp.float32),
    mesh=_mesh,
    compiler_params=_sc_params,
    scratch_types=_scatter_scratch,
)


BR = 3136


def _mm_bias_kern(x_ref, w_ref, b_ref, o_ref):
    o_ref[...] = (jnp.dot(x_ref[...], w_ref[...],
                          preferred_element_type=jnp.float32)
                  + b_ref[...])


def _mm_bias(x, w, b):
    p = w.shape[1]
    return pl.pallas_call(
        _mm_bias_kern,
        grid=(NPAD // BR,),
        in_specs=[
            pl.BlockSpec((BR, 128), lambda i: (i, 0)),
            pl.BlockSpec((128, p), lambda i: (0, 0)),
            pl.BlockSpec((1, p), lambda i: (0, 0)),
        ],
        out_specs=pl.BlockSpec((BR, p), lambda i: (i, 0)),
        out_shape=jax.ShapeDtypeStruct((NPAD, p), jnp.float32),
    )(x, w, b.reshape(1, p))


def _combine_kern(s1_ref, s2_ref, s3_ref, cnt_ref, emb_ref, o_ref):
    rc = 1.0 / jnp.maximum(cnt_ref[...], 1.0)
    o_ref[...] = 0.5 * (jnp.maximum(s1_ref[...] * rc, 0.0)
                        + jnp.maximum(s2_ref[...] * s3_ref[...], 0.0)
                        + emb_ref[...])


def _combine(s1, s2, s3, cnt, emb):
    return pl.pallas_call(
        _combine_kern,
        grid=(NPAD // BR,),
        in_specs=[
            pl.BlockSpec((BR, 128), lambda i: (i, 0)),
            pl.BlockSpec((BR, 128), lambda i: (i, 0)),
            pl.BlockSpec((BR, 128), lambda i: (i, 0)),
            pl.BlockSpec((BR, 1), lambda i: (i, 0)),
            pl.BlockSpec((BR, 128), lambda i: (i, 0)),
        ],
        out_specs=pl.BlockSpec((BR, 128), lambda i: (i, 0)),
        out_shape=jax.ShapeDtypeStruct((NPAD, 128), jnp.float32),
    )(s1, s2, s3, cnt.reshape(NPAD, 1), emb)


def kernel(x, params, b0_cor, b0_sim, b1_cor, b1_sim):
    xpad = jnp.pad(x, ((0, NPAD - N), (0, 0)))
    names = ("b0_cor", "b0_sim", "b1_cor", "b1_sim")
    arrs = (b0_cor, b0_sim, b1_cor, b1_sim)
    epad = {nm: jnp.pad(a, ((0, 0), (0, EPAD - E)), constant_values=NPAD - 1)
            for nm, a in zip(names, arrs)}

    src_all = jnp.concatenate([epad[nm][0] for nm in names])
    dst_all = jnp.concatenate([epad[nm][1] for nm in names])
    cnt_flat, pk_flat = _sc_prep(src_all, dst_all)
    cnt = cnt_flat.reshape(4, NPAD)
    pk = {nm: pk_flat[i * EPAD:(i + 1) * EPAD] for i, nm in enumerate(names)}

    cnt_row = {("cor", 0): 0, ("sim", 0): 1, ("cor", 1): 2, ("sim", 1): 3}
    blocks = [("b0_cor", "b0_sim"), ("b1_cor", "b1_sim")]

    outs = []
    for mode in ("cor", "sim"):
        pm = params[mode]
        emb = _mm_bias(xpad, pm["fcin_w"], pm["fcin_b"])
        for li, (cor_nm, sim_nm) in enumerate(blocks):
            em_nm, eo_nm = ((cor_nm, sim_nm) if mode == "cor"
                            else (sim_nm, cor_nm))
            patt = pm["l1_att"] if li == 0 else pm["l2_att"]
            pagg = pm["l1_agg"] if li == 0 else pm["l2_agg"]
            ea, ia = patt["e_att"], patt["i_att"]
            ds_w = (ea[:D, 0] - ea[:D, 1])[:, None]
            dd_w = (ea[D:, 0] - ea[D:, 1])[:, None]
            wcat = jnp.concatenate(
                [pagg["W"], ds_w, dd_w, ia[:D, 0:1], ia[D:, 0:1],
                 jnp.zeros((D, 124), jnp.float32)], axis=1)
            bcat = jnp.concatenate([pagg["b"], jnp.zeros((128,), jnp.float32)])
            y = _mm_bias(emb, wcat, bcat)
            h = y[:, :D]
            gs, gd = y[:, D], y[:, D + 1]
            t1, t2 = y[:, D + 2], y[:, D + 3]

            scm, sco = _sc_scales(pk[em_nm], pk[eo_nm], gs, gd, t1, t2)
            s1, s2 = _sc_scatter12(h, emb, pk[em_nm], scm)
            s3 = _sc_scatter3(emb, pk[eo_nm], sco)
            emb = _combine(s1, s2, s3, cnt[cnt_row[(mode, li)]], emb)
        outs.append(_mm_bias(emb, pm["fcout_w"], pm["fcout_b"])[:N])
    return tuple(outs)


# speedup vs baseline: 2.0625x; 1.0077x over previous
import jax
import jax.numpy as jnp
from jax import lax
from jax.experimental import pallas as pl
from jax.experimental.pallas import tpu as pltpu
from jax.experimental.pallas import tpu_sc as plsc

N = 50000
D = 128
E = 200000

NPAD = 50176
EPAD = 200192
CH = 6272
ACCR = 6400
STRIPE = EPAD // 16
HSTRIPE = STRIPE // 2
ESH = EPAD // 32
K = 16
PAIR = 2 * K

_mesh = plsc.VectorSubcoreMesh(core_axis_name="c", subcore_axis_name="s")
_sc_params = pltpu.CompilerParams(needs_layout_passes=False)


def _sig(z):
    return 1.0 / (1.0 + jnp.exp(-z))


def _unpack_src(p16):
    return p16 & 0xFFFF


def _unpack_dst(p16):
    return lax.shift_right_logical(p16, 16)


def _sc_prep_body(src_all, dst_all, cnt, pk, acc_l, se_v, de_v, pk_v, tmp,
                  res, stage):
    c = lax.axis_index("c")
    s = lax.axis_index("s")
    zero16 = jnp.zeros((16,), jnp.float32)
    ones16 = jnp.ones((16,), jnp.float32)

    for a in range(2):
        arr = c * 2 + a

        def _z(i, _):
            for q in range(8):
                acc_l[pl.ds((i * 8 + q) * 16, 16)] = zero16
            return 0
        lax.fori_loop(0, NPAD // 128, _z, 0)

        def _scan(i, _):
            s16 = se_v[pl.ds(i * 16, 16)]
            d16 = de_v[pl.ds(i * 16, 16)]
            pk_v[pl.ds(i * 16, 16)] = (d16 << 16) | s16
            plsc.addupdate_scatter(acc_l, [d16], ones16)
            return 0

        for h in range(2):
            base = arr * EPAD + s * STRIPE + h * HSTRIPE
            pltpu.sync_copy(src_all.at[pl.ds(base, HSTRIPE)], se_v)
            pltpu.sync_copy(dst_all.at[pl.ds(base, HSTRIPE)], de_v)
            lax.fori_loop(0, HSTRIPE // 16, _scan, 0)
            pltpu.sync_copy(pk_v, pk.at[pl.ds(base, HSTRIPE)])

        pltpu.sync_copy(acc_l, stage.at[pl.ds(s * NPAD, NPAD)])
        plsc.subcore_barrier()

        colbase = s * (NPAD // 16)
        pltpu.sync_copy(stage.at[pl.ds(colbase, NPAD // 16)], res)
        for j in range(1, 16):
            pltpu.sync_copy(stage.at[pl.ds(j * NPAD + colbase, NPAD // 16)],
                            tmp)

            def _add(i, _):
                for q in range(4):
                    o = (i * 4 + q) * 16
                    res[pl.ds(o, 16)] = (res[pl.ds(o, 16)]
                                         + tmp[pl.ds(o, 16)])
                return 0
            lax.fori_loop(0, NPAD // 16 // 64, _add, 0)
        pltpu.sync_copy(res, cnt.at[pl.ds(arr * NPAD + colbase, NPAD // 16)])
        plsc.subcore_barrier()


_sc_prep = pl.kernel(
    _sc_prep_body,
    out_type=(jax.ShapeDtypeStruct((4 * NPAD,), jnp.float32),
              jax.ShapeDtypeStruct((4 * EPAD,), jnp.int32)),
    mesh=_mesh,
    compiler_params=_sc_params,
    scratch_types=[
        pltpu.VMEM((NPAD,), jnp.float32),
        pltpu.VMEM((HSTRIPE,), jnp.int32),
        pltpu.VMEM((HSTRIPE,), jnp.int32),
        pltpu.VMEM((HSTRIPE,), jnp.int32),
        pltpu.VMEM((NPAD // 16,), jnp.float32),
        pltpu.VMEM((NPAD // 16,), jnp.float32),
        pltpu.VMEM_SHARED((16 * NPAD,), jnp.float32),
    ],
)


def _sc_scales_body(pk_em, pk_eo, gs, gd, t1, t2, scm, sco,
                    g1_v, g2_v, pk_v, out_v):
    c = lax.axis_index("c")
    s = lax.axis_index("s")
    wid = s * 2 + c
    ebase = wid * ESH

    for part, (ga, gb, pkr, outr) in enumerate(
            (((gs, gd, pk_em, scm)), (t1, t2, pk_eo, sco))):
        pltpu.sync_copy(ga, g1_v)
        pltpu.sync_copy(gb, g2_v)
        pltpu.sync_copy(pkr.at[pl.ds(ebase, ESH)], pk_v)

        def _lp(i, _):
            p16 = pk_v[pl.ds(i * 16, 16)]
            z = (plsc.load_gather(g1_v, [_unpack_src(p16)])
                 + plsc.load_gather(g2_v, [_unpack_dst(p16)]))
            out_v[pl.ds(i * 16, 16)] = _sig(z)
            return 0
        lax.fori_loop(0, ESH // 16, _lp, 0)
        pltpu.sync_copy(out_v, outr.at[pl.ds(ebase, ESH)])


_sc_scales = pl.kernel(
    _sc_scales_body,
    out_type=(jax.ShapeDtypeStruct((EPAD,), jnp.float32),
              jax.ShapeDtypeStruct((EPAD,), jnp.float32)),
    mesh=_mesh,
    compiler_params=_sc_params,
    scratch_types=[
        pltpu.VMEM((NPAD,), jnp.float32),
        pltpu.VMEM((NPAD,), jnp.float32),
        pltpu.VMEM((ESH,), jnp.int32),
        pltpu.VMEM((ESH,), jnp.float32),
    ],
)


def _zero_acc(acc, zacc_v, s, sem):
    for z in range(12):
        pltpu.async_copy(zacc_v, acc.at[pl.ds(s * 400 + z * 32, 32)], sem)
    pltpu.async_copy(zacc_v.at[pl.ds(0, 16)],
                     acc.at[pl.ds(s * 400 + 384, 16)], sem)
    for z in range(12):
        pltpu.make_async_copy(zacc_v, acc.at[pl.ds(s * 400 + z * 32, 32)],
                              sem).wait()
    pltpu.make_async_copy(zacc_v.at[pl.ds(0, 16)],
                          acc.at[pl.ds(s * 400 + 384, 16)], sem).wait()


def _scan_chunk(pk, sc, pe_v, sce_v, cp_v, cc_v, s, lo):
    def _scan(i, m):
        p16 = pe_v[pl.ds(i * 16, 16)]
        f16 = sce_v[pl.ds(i * 16, 16)]
        off16 = _unpack_dst(p16) - lo
        msk = (off16 >= 0) & (off16 < CH)
        plsc.store_compressed(cp_v.at[pl.ds(m, 16)], p16, mask=msk)
        plsc.store_compressed(cc_v.at[pl.ds(m, 16)], f16, mask=msk)
        return m + jnp.sum(msk.astype(jnp.int32))

    m = 0
    for h in range(2):
        base = s * STRIPE + h * HSTRIPE
        pltpu.sync_copy(pk.at[pl.ds(base, HSTRIPE)], pe_v)
        pltpu.sync_copy(sc.at[pl.ds(base, HSTRIPE)], sce_v)
        m = lax.fori_loop(0, HSTRIPE // 16, _scan, m)

    dummp = jnp.full((16,), (lo + CH + 8) << 16, jnp.int32)
    zf16 = jnp.zeros((16,), jnp.float32)
    for t in range(PAIR // 16):
        cp_v[pl.ds(m + t * 16, 16)] = dummp
        cc_v[pl.ds(m + t * 16, 16)] = zf16
    return jnp.maximum((m + PAIR - 1) // PAIR, 1)


def _flush(tab, comp, npairs, lo, cp_v, cc_v, gidx0, gidx1, sidx0, sidx1,
           rows0, rows1, acc, sem_g0, sem_g1, sem_s0, sem_s1):
    def stage_g(j, gidx, rows, sem):
        for q in range(K // 16):
            p16 = cp_v[pl.ds(j * K + q * 16, 16)]
            gidx[q * 16:(q + 1) * 16] = _unpack_src(p16)
        pltpu.async_copy(tab.at[gidx], rows, sem)

    def prep_sidx(j, sidx):
        for q in range(K // 16):
            p16 = cp_v[pl.ds(j * K + q * 16, 16)]
            sidx[q * 16:(q + 1) * 16] = _unpack_dst(p16) - lo

    def scale_rows(j, rows):
        @plsc.parallel_loop(0, K // 8, 1, unroll=2)
        def _rb(r8):
            sv = cc_v[pl.ds(j * K + r8 * 8, 16)]
            if comp:
                sv = 1.0 - sv
            for dr in range(8):
                scv = sv[dr]
                r = r8 * 8 + dr
                for q in range(8):
                    rows[r, q * 16:(q + 1) * 16] = (
                        scv * rows[r, q * 16:(q + 1) * 16])

    stage_g(0, gidx0, rows0, sem_g0)

    def _pair(t, _):
        j0 = 2 * t

        @pl.when(t > 0)
        def _():
            pltpu.make_async_copy(rows1, acc.at[sidx1], sem_s1).wait()
        stage_g(j0 + 1, gidx1, rows1, sem_g1)
        pltpu.make_async_copy(tab.at[gidx0], rows0, sem_g0).wait()
        prep_sidx(j0, sidx0)
        scale_rows(j0, rows0)
        pltpu.async_copy(rows0, acc.at[sidx0], sem_s0, add=True)

        pltpu.make_async_copy(tab.at[gidx1], rows1, sem_g1).wait()
        prep_sidx(j0 + 1, sidx1)
        scale_rows(j0 + 1, rows1)

        @pl.when(t + 1 < npairs)
        def _():
            pltpu.make_async_copy(rows0, acc.at[sidx0], sem_s0).wait()
            stage_g(j0 + 2, gidx0, rows0, sem_g0)
        pltpu.async_copy(rows1, acc.at[sidx1], sem_s1, add=True)
        return 0

    lax.fori_loop(0, npairs, _pair, 0)
    pltpu.make_async_copy(rows0, acc.at[sidx0], sem_s0).wait()
    pltpu.make_async_copy(rows1, acc.at[sidx1], sem_s1).wait()


def _dump(acc, out, s, lo):
    pltpu.sync_copy(acc.at[pl.ds(s * 392, 392)],
                    out.at[pl.ds(lo + s * 392, 392)])


def _sc_scatter12_body(tab1, tab2, pk, sc, out1, out2,
                       pe_v, sce_v, cp_v, cc_v, gidx0, gidx1, sidx0, sidx1,
                       rows0, rows1, zacc_v, acc,
                       sem_g0, sem_g1, sem_s0, sem_s1):
    c = lax.axis_index("c")
    s = lax.axis_index("s")
    zero16 = jnp.zeros((16,), jnp.float32)

    def _z(i, _):
        for q in range(8):
            zacc_v[i, q * 16:(q + 1) * 16] = zero16
        return 0
    lax.fori_loop(0, 32, _z, 0)

    bufs = (cp_v, cc_v, gidx0, gidx1, sidx0, sidx1, rows0, rows1, acc,
            sem_g0, sem_g1, sem_s0, sem_s1)
    for k in range(4):
        lo = (c * 4 + k) * CH
        _zero_acc(acc, zacc_v, s, sem_g0)
        plsc.subcore_barrier()
        nbat = _scan_chunk(pk, sc, pe_v, sce_v, cp_v, cc_v, s, lo)
        for rep, (tab, out, comp) in enumerate(((tab1, out1, False),
                                                (tab2, out2, True))):
            _flush(tab, comp, nbat, lo, *bufs)
            plsc.subcore_barrier()
            _dump(acc, out, s, lo)
            plsc.subcore_barrier()
            if rep == 0:
                _zero_acc(acc, zacc_v, s, sem_g0)
                plsc.subcore_barrier()


def _sc_scatter3_body(tab, pk, sc, out,
                      pe_v, sce_v, cp_v, cc_v, gidx0, gidx1, sidx0, sidx1,
                      rows0, rows1, zacc_v, acc,
                      sem_g0, sem_g1, sem_s0, sem_s1):
    c = lax.axis_index("c")
    s = lax.axis_index("s")
    zero16 = jnp.zeros((16,), jnp.float32)

    def _z(i, _):
        for q in range(8):
            zacc_v[i, q * 16:(q + 1) * 16] = zero16
        return 0
    lax.fori_loop(0, 32, _z, 0)

    for k in range(4):
        lo = (c * 4 + k) * CH
        _zero_acc(acc, zacc_v, s, sem_g0)
        plsc.subcore_barrier()
        nbat = _scan_chunk(pk, sc, pe_v, sce_v, cp_v, cc_v, s, lo)
        _flush(tab, False, nbat, lo, cp_v, cc_v, gidx0, gidx1, sidx0,
               sidx1, rows0, rows1, acc, sem_g0, sem_g1, sem_s0, sem_s1)
        plsc.subcore_barrier()
        _dump(acc, out, s, lo)
        plsc.subcore_barrier()


_scatter_scratch = [
    pltpu.VMEM((HSTRIPE,), jnp.int32),
    pltpu.VMEM((HSTRIPE,), jnp.float32),
    pltpu.VMEM((STRIPE + 2 * PAIR,), jnp.int32),
    pltpu.VMEM((STRIPE + 2 * PAIR,), jnp.float32),
    pltpu.VMEM((K,), jnp.int32),
    pltpu.VMEM((K,), jnp.int32),
    pltpu.VMEM((K,), jnp.int32),
    pltpu.VMEM((K,), jnp.int32),
    pltpu.VMEM((K, 128), jnp.float32),
    pltpu.VMEM((K, 128), jnp.float32),
    pltpu.VMEM((32, 128), jnp.float32),
    pltpu.VMEM_SHARED((ACCR, 128), jnp.float32),
    pltpu.SemaphoreType.DMA,
    pltpu.SemaphoreType.DMA,
    pltpu.SemaphoreType.DMA,
    pltpu.SemaphoreType.DMA,
]

_sc_scatter12 = pl.kernel(
    _sc_scatter12_body,
    out_type=(jax.ShapeDtypeStruct((NPAD, 128), jnp.float32),
              jax.ShapeDtypeStruct((NPAD, 128), jnp.float32)),
    mesh=_mesh,
    compiler_params=_sc_params,
    scratch_types=_scatter_scratch,
)

_sc_scatter3 = pl.kernel(
    _sc_scatter3_body,
    out_type=jax.ShapeDtypeStruct((NPAD, 128), jnp.float32),
    mesh=_mesh,
    compiler_params=_sc_params,
    scratch_types=_scatter_scratch,
)


BR = 3136


def _mm_bias_kern(x_ref, w_ref, b_ref, o_ref):
    o_ref[...] = (jnp.dot(x_ref[...], w_ref[...],
                          preferred_element_type=jnp.float32)
                  + b_ref[...])


def _mm_bias(x, w, b):
    p = w.shape[1]
    return pl.pallas_call(
        _mm_bias_kern,
        grid=(NPAD // BR,),
        in_specs=[
            pl.BlockSpec((BR, 128), lambda i: (i, 0)),
            pl.BlockSpec((128, p), lambda i: (0, 0)),
            pl.BlockSpec((1, p), lambda i: (0, 0)),
        ],
        out_specs=pl.BlockSpec((BR, p), lambda i: (i, 0)),
        out_shape=jax.ShapeDtypeStruct((NPAD, p), jnp.float32),
    )(x, w, b.reshape(1, p))


def _combine_kern(s1_ref, s2_ref, s3_ref, cnt_ref, emb_ref, o_ref):
    rc = 1.0 / jnp.maximum(cnt_ref[...], 1.0)
    o_ref[...] = 0.5 * (jnp.maximum(s1_ref[...] * rc, 0.0)
                        + jnp.maximum(s2_ref[...] * s3_ref[...], 0.0)
                        + emb_ref[...])


def _combine(s1, s2, s3, cnt, emb):
    return pl.pallas_call(
        _combine_kern,
        grid=(NPAD // BR,),
        in_specs=[
            pl.BlockSpec((BR, 128), lambda i: (i, 0)),
            pl.BlockSpec((BR, 128), lambda i: (i, 0)),
            pl.BlockSpec((BR, 128), lambda i: (i, 0)),
            pl.BlockSpec((BR, 1), lambda i: (i, 0)),
            pl.BlockSpec((BR, 128), lambda i: (i, 0)),
        ],
        out_specs=pl.BlockSpec((BR, 128), lambda i: (i, 0)),
        out_shape=jax.ShapeDtypeStruct((NPAD, 128), jnp.float32),
    )(s1, s2, s3, cnt.reshape(NPAD, 1), emb)


def kernel(x, params, b0_cor, b0_sim, b1_cor, b1_sim):
    xpad = jnp.pad(x, ((0, NPAD - N), (0, 0)))
    names = ("b0_cor", "b0_sim", "b1_cor", "b1_sim")
    arrs = (b0_cor, b0_sim, b1_cor, b1_sim)
    epad = {nm: jnp.pad(a, ((0, 0), (0, EPAD - E)), constant_values=NPAD - 1)
            for nm, a in zip(names, arrs)}

    src_all = jnp.concatenate([epad[nm][0] for nm in names])
    dst_all = jnp.concatenate([epad[nm][1] for nm in names])
    cnt_flat, pk_flat = _sc_prep(src_all, dst_all)
    cnt = cnt_flat.reshape(4, NPAD)
    pk = {nm: pk_flat[i * EPAD:(i + 1) * EPAD] for i, nm in enumerate(names)}

    cnt_row = {("cor", 0): 0, ("sim", 0): 1, ("cor", 1): 2, ("sim", 1): 3}
    blocks = [("b0_cor", "b0_sim"), ("b1_cor", "b1_sim")]

    outs = []
    for mode in ("cor", "sim"):
        pm = params[mode]
        emb = _mm_bias(xpad, pm["fcin_w"], pm["fcin_b"])
        for li, (cor_nm, sim_nm) in enumerate(blocks):
            em_nm, eo_nm = ((cor_nm, sim_nm) if mode == "cor"
                            else (sim_nm, cor_nm))
            patt = pm["l1_att"] if li == 0 else pm["l2_att"]
            pagg = pm["l1_agg"] if li == 0 else pm["l2_agg"]
            ea, ia = patt["e_att"], patt["i_att"]
            ds_w = (ea[:D, 0] - ea[:D, 1])[:, None]
            dd_w = (ea[D:, 0] - ea[D:, 1])[:, None]
            wcat = jnp.concatenate(
                [pagg["W"], ds_w, dd_w, ia[:D, 0:1], ia[D:, 0:1],
                 jnp.zeros((D, 124), jnp.float32)], axis=1)
            bcat = jnp.concatenate([pagg["b"], jnp.zeros((128,), jnp.float32)])
            y = _mm_bias(emb, wcat, bcat)
            h = y[:, :D]
            gs, gd = y[:, D], y[:, D + 1]
            t1, t2 = y[:, D + 2], y[:, D + 3]

            scm, sco = _sc_scales(pk[em_nm], pk[eo_nm], gs, gd, t1, t2)
            s1, s2 = _sc_scatter12(h, emb, pk[em_nm], scm)
            s3 = _sc_scatter3(emb, pk[eo_nm], sco)
            emb = _combine(s1, s2, s3, cnt[cnt_row[(mode, li)]], emb)
        outs.append(_mm_bias(emb, pm["fcout_w"], pm["fcout_b"])[:N])
    return tuple(outs)
